# baseline (speedup 1.0000x reference)
import functools

import jax
import jax.numpy as jnp
from jax.experimental import pallas as pl

BATCH = 4096
IN_DIM = 768
E_DIM = 32
NUM_LEVELS = 4
K = 8192
BETA = 0.25
TILE = 512


def _fused_kernel(x_ref, We0_ref, be0_ref, We1_ref, be1_ref, We2_ref, be2_ref,
                  Wd0_ref, bd0_ref, Wd1_ref, bd1_ref, Wd2_ref, bd2_ref,
                  cbT_ref, out_ref, idx_ref, lsum_ref):
    f32 = jnp.float32
    dot = functools.partial(jax.lax.dot_general, preferred_element_type=f32)

    @pl.when(pl.program_id(0) == 0)
    def _init():
        lsum_ref[...] = jnp.zeros_like(lsum_ref)

    h = jnp.maximum(dot(x_ref[...], We0_ref[...], (((1,), (0,)), ((), ()))) + be0_ref[...], 0.0)
    h = jnp.maximum(dot(h, We1_ref[...], (((1,), (0,)), ((), ()))) + be1_ref[...], 0.0)
    z = dot(h, We2_ref[...], (((1,), (0,)), ((), ()))) + be2_ref[...]

    r = z
    xq = jnp.zeros_like(z)
    for l in range(NUM_LEVELS):
        cbT = cbT_ref[l]
        csq = jnp.sum(cbT * cbT, axis=0, keepdims=True)
        s = csq - 2.0 * dot(r, cbT, (((1,), (0,)), ((), ())))
        m = jnp.min(s, axis=1, keepdims=True)
        lane = jax.lax.broadcasted_iota(jnp.int32, s.shape, 1)
        idx = jnp.min(jnp.where(s == m, lane, K), axis=1, keepdims=True)
        idx_ref[:, l:l + 1] = idx
        onehot = (lane == idx).astype(f32)
        q = dot(onehot, cbT, (((1,), (1,)), ((), ())),
                precision=jax.lax.Precision.HIGHEST)
        t = q - r
        lsum_ref[l:l + 1, :] += jnp.sum(t * t)
        q_st = r + t
        xq = xq + q_st
        r = r - q_st

    h = jnp.maximum(dot(xq, Wd0_ref[...], (((1,), (0,)), ((), ()))) + bd0_ref[...], 0.0)
    h = jnp.maximum(dot(h, Wd1_ref[...], (((1,), (0,)), ((), ()))) + bd1_ref[...], 0.0)
    out_ref[...] = dot(h, Wd2_ref[...], (((1,), (0,)), ((), ()))) + bd2_ref[...]


def kernel(x, We0, be0, We1, be1, We2, be2, Wd0, bd0, Wd1, bd1, Wd2, bd2, codebooks):
    cbT = jnp.transpose(codebooks, (0, 2, 1))
    grid = (BATCH // TILE,)

    def row_block(d1):
        return pl.BlockSpec((TILE, d1), lambda i: (i, 0))

    def full(shape):
        return pl.BlockSpec(shape, lambda i: (0,) * len(shape))

    out, idx, lsum = pl.pallas_call(
        _fused_kernel,
        grid=grid,
        in_specs=[
            row_block(IN_DIM),
            full((IN_DIM, 512)), full((1, 512)),
            full((512, 128)), full((1, 128)),
            full((128, E_DIM)), full((1, E_DIM)),
            full((E_DIM, 128)), full((1, 128)),
            full((128, 512)), full((1, 512)),
            full((512, IN_DIM)), full((1, IN_DIM)),
            full((NUM_LEVELS, E_DIM, K)),
        ],
        out_specs=[
            row_block(IN_DIM),
            pl.BlockSpec((TILE, NUM_LEVELS), lambda i: (i, 0)),
            full((NUM_LEVELS, 128)),
        ],
        out_shape=[
            jax.ShapeDtypeStruct((BATCH, IN_DIM), jnp.float32),
            jax.ShapeDtypeStruct((BATCH, NUM_LEVELS), jnp.int32),
            jax.ShapeDtypeStruct((NUM_LEVELS, 128), jnp.float32),
        ],
    )(x, We0, be0.reshape(1, -1), We1, be1.reshape(1, -1), We2, be2.reshape(1, -1),
      Wd0, bd0.reshape(1, -1), Wd1, bd1.reshape(1, -1), Wd2, bd2.reshape(1, -1), cbT)

    per_level = lsum[:, 0] / (BATCH * E_DIM)
    rq_loss = jnp.mean((1.0 + BETA) * per_level)
    orth_loss = jnp.zeros((), dtype=jnp.float32)
    return (out, rq_loss, orth_loss, idx)


# speedup vs baseline: 1.5176x; 1.5176x over previous
import functools

import jax
import jax.numpy as jnp
from jax.experimental import pallas as pl

BATCH = 4096
IN_DIM = 768
E_DIM = 32
NUM_LEVELS = 4
K = 8192
BETA = 0.25
TILE = 512


def _fused_kernel(x_ref, We0_ref, be0_ref, We1_ref, be1_ref, We2_ref, be2_ref,
                  Wd0_ref, bd0_ref, Wd1_ref, bd1_ref, Wd2_ref, bd2_ref,
                  cbT_ref, cb0_ref, cb1_ref, cb2_ref, out_ref, idx_ref, lsum_ref):
    f32 = jnp.float32
    dot = functools.partial(jax.lax.dot_general, preferred_element_type=f32)

    @pl.when(pl.program_id(0) == 0)
    def _init():
        lsum_ref[...] = jnp.zeros_like(lsum_ref)

    h = jnp.maximum(dot(x_ref[...], We0_ref[...], (((1,), (0,)), ((), ()))) + be0_ref[...], 0.0)
    h = jnp.maximum(dot(h, We1_ref[...], (((1,), (0,)), ((), ()))) + be1_ref[...], 0.0)
    z = dot(h, We2_ref[...], (((1,), (0,)), ((), ()))) + be2_ref[...]

    r = z
    xq = jnp.zeros_like(z)
    lane = jax.lax.broadcasted_iota(jnp.int32, (x_ref.shape[0], K), 1)
    for l in range(NUM_LEVELS):
        cbT = cbT_ref[l]
        csq = jnp.sum(cbT * cbT, axis=0, keepdims=True)
        s = csq + dot(-2.0 * r, cbT, (((1,), (0,)), ((), ())))
        m = jnp.min(s, axis=1, keepdims=True)
        idx = jnp.min(jnp.where(s == m, lane, K), axis=1, keepdims=True)
        idx_ref[:, l:l + 1] = idx
        onehot = (lane == idx).astype(jnp.bfloat16)
        dn = (((1,), (1,)), ((), ()))
        q = (dot(onehot, cb0_ref[l], dn) + dot(onehot, cb1_ref[l], dn)) \
            + dot(onehot, cb2_ref[l], dn)
        t = q - r
        lsum_ref[l:l + 1, :] += jnp.sum(t * t)
        q_st = r + t
        xq = xq + q_st
        r = r - q_st

    h = jnp.maximum(dot(xq, Wd0_ref[...], (((1,), (0,)), ((), ()))) + bd0_ref[...], 0.0)
    h = jnp.maximum(dot(h, Wd1_ref[...], (((1,), (0,)), ((), ()))) + bd1_ref[...], 0.0)
    out_ref[...] = dot(h, Wd2_ref[...], (((1,), (0,)), ((), ()))) + bd2_ref[...]


def kernel(x, We0, be0, We1, be1, We2, be2, Wd0, bd0, Wd1, bd1, Wd2, bd2, codebooks):
    cbT = jnp.transpose(codebooks, (0, 2, 1))
    cb0 = cbT.astype(jnp.bfloat16)
    r1 = cbT - cb0.astype(jnp.float32)
    cb1 = r1.astype(jnp.bfloat16)
    cb2 = (r1 - cb1.astype(jnp.float32)).astype(jnp.bfloat16)
    grid = (BATCH // TILE,)

    def row_block(d1):
        return pl.BlockSpec((TILE, d1), lambda i: (i, 0))

    def full(shape):
        return pl.BlockSpec(shape, lambda i: (0,) * len(shape))

    out, idx, lsum = pl.pallas_call(
        _fused_kernel,
        grid=grid,
        in_specs=[
            row_block(IN_DIM),
            full((IN_DIM, 512)), full((1, 512)),
            full((512, 128)), full((1, 128)),
            full((128, E_DIM)), full((1, E_DIM)),
            full((E_DIM, 128)), full((1, 128)),
            full((128, 512)), full((1, 512)),
            full((512, IN_DIM)), full((1, IN_DIM)),
            full((NUM_LEVELS, E_DIM, K)),
            full((NUM_LEVELS, E_DIM, K)),
            full((NUM_LEVELS, E_DIM, K)),
            full((NUM_LEVELS, E_DIM, K)),
        ],
        out_specs=[
            row_block(IN_DIM),
            pl.BlockSpec((TILE, NUM_LEVELS), lambda i: (i, 0)),
            full((NUM_LEVELS, 128)),
        ],
        out_shape=[
            jax.ShapeDtypeStruct((BATCH, IN_DIM), jnp.float32),
            jax.ShapeDtypeStruct((BATCH, NUM_LEVELS), jnp.int32),
            jax.ShapeDtypeStruct((NUM_LEVELS, 128), jnp.float32),
        ],
    )(x, We0, be0.reshape(1, -1), We1, be1.reshape(1, -1), We2, be2.reshape(1, -1),
      Wd0, bd0.reshape(1, -1), Wd1, bd1.reshape(1, -1), Wd2, bd2.reshape(1, -1),
      cbT, cb0, cb1, cb2)

    per_level = lsum[:, 0] / (BATCH * E_DIM)
    rq_loss = jnp.mean((1.0 + BETA) * per_level)
    orth_loss = jnp.zeros((), dtype=jnp.float32)
    return (out, rq_loss, orth_loss, idx)
